# baseline (speedup 1.0000x reference)
import jax
import jax.numpy as jnp
from jax import lax
from jax.experimental import pallas as pl
from jax.experimental.pallas import tpu as pltpu
from jax.experimental.pallas import tpu_sc as plsc

TOPK = 5
IGN = -100
V = 100000
N = 1024
L = 16
NW = 32
G = 4
OSC = 49280
CWA = 6272
NA = 8
ORG = OSC + CWA * NA
CRG = 512
OTL = ORG + CRG
CTL = V - OTL
CVA = CWA // L
CVR = CRG // L
CVT = CTL // L
UNROLL = 8


def _gather_body(mat_hbm, targ_hbm, tv_hbm, targv, wbuf, stage, sem_w):
    wid = lax.axis_index("s") * 2 + lax.axis_index("c")
    base = wid * (8 * G)
    pltpu.sync_copy(targ_hbm.at[pl.ds(base, 8 * G)], targv)

    for j in range(8 * G):
        v16 = targv[pl.ds((j // L) * L, L)]
        tj = jnp.clip(v16[j % L], 0, V - 1)
        a128 = pl.multiple_of((tj // 128) * 128, 128)
        rs = pl.multiple_of(base + (j // 8) * 8, 8)
        pltpu.async_copy(mat_hbm.at[pl.ds(rs, 8), pl.ds(a128, 128)],
                         wbuf.at[j], sem_w)
    for j in range(8 * G):
        pltpu.make_async_copy(mat_hbm.at[pl.ds(0, 8), pl.ds(0, 128)],
                              wbuf.at[j], sem_w).wait()
    for j in range(8 * G):
        tidx16 = plsc.load_gather(targv, [jnp.full((L,), j, jnp.int32)])
        tcv = jnp.clip(tidx16, 0, V - 1)
        tval16 = plsc.load_gather(
            wbuf, [jnp.full((L,), j, jnp.int32),
                   jnp.full((L,), j % 8, jnp.int32), tcv % 128])
        stage[pl.ds(j * L, L)] = tval16
    pltpu.sync_copy(stage, tv_hbm.at[pl.ds(base * L, 8 * G * L)])


def _count_body(mat_hbm, targ_hbm, tv_hbm, out_hbm, targv, tvv, accb, pcntb,
                bufa0, bufa1, bufr, buft, rblock, sem_c):
    wid = lax.axis_index("s") * 2 + lax.axis_index("c")
    base = wid * (8 * G)

    pltpu.sync_copy(targ_hbm.at[pl.ds(base, 8 * G)], targv)
    pltpu.sync_copy(tv_hbm.at[pl.ds(base * L, 8 * G * L)], tvv)

    lane = lax.iota(jnp.int32, L)
    zero16 = jnp.zeros((L,), jnp.int32)
    bufs = (bufa0, bufa1)
    for j in range(8 * G):
        accb[pl.ds(j * L, L)] = zero16
        pcntb[pl.ds(j * L, L)] = zero16

    pltpu.async_copy(mat_hbm.at[pl.ds(base, 8), pl.ds(OSC, CWA)], bufa0, sem_c)

    def wait_dma(buf, w):
        pltpu.make_async_copy(mat_hbm.at[pl.ds(0, 8), pl.ds(0, w)],
                              buf, sem_c).wait()

    def count_segment(buf, r, j, o, cv):
        tval16 = tvv[pl.ds(j * L, L)]
        tidx16 = plsc.load_gather(targv, [jnp.full((L,), j, jnp.int32)])
        tc = jnp.max(jnp.clip(tidx16, 0, V - 1))
        s = jnp.clip(tc - o, 0, cv * L)
        fs = s // L
        pcnt = pcntb[pl.ds(j * L, L)]
        acc = accb[pl.ds(j * L, L)]

        @plsc.parallel_loop(0, fs, unroll=UNROLL, carry=pcnt)
        def ge_loop(i, a):
            x = buf[r, pl.ds(i * L, L)]
            return a + plsc.all_reduce_population_count(x >= tval16)
        pcnt = ge_loop

        fm = jnp.minimum(fs, cv - 1)
        x = buf[r, pl.ds(fm * L, L)]
        posv = jnp.full((L,), o + fm * L, jnp.int32) + lane
        m = (x > tval16) | ((x == tval16) & (posv < tidx16))
        m = m & (jnp.full((L,), fs, jnp.int32) < cv)
        acc = acc + jnp.where(m, 1, 0).astype(jnp.int32)

        @plsc.parallel_loop(fs + 1, cv, unroll=UNROLL, carry=pcnt)
        def gt_loop(i, a):
            x = buf[r, pl.ds(i * L, L)]
            return a + plsc.all_reduce_population_count(x > tval16)
        pcnt = gt_loop

        pcntb[pl.ds(j * L, L)] = pcnt
        accb[pl.ds(j * L, L)] = acc

    def group_body(g, _):
        rs_g = pl.multiple_of(base + g * 8, 8)

        def pair_body(p, _2):
            for k in range(2):
                c = 2 * p + k
                wait_dma(bufs[k], CWA)
                o_next = pl.multiple_of(OSC + (c + 1) * CWA, 128)

                @pl.when(c < NA - 1)
                def _3():
                    pltpu.async_copy(
                        mat_hbm.at[pl.ds(rs_g, 8), pl.ds(o_next, CWA)],
                        bufs[1 - k], sem_c)

                @pl.when(c == NA - 1)
                def _3b():
                    pltpu.async_copy(
                        mat_hbm.at[pl.ds(rs_g, 8), pl.ds(ORG, CRG)],
                        bufr, sem_c)
                    pltpu.async_copy(
                        mat_hbm.at[pl.ds(rs_g, 8), pl.ds(OTL, CTL)],
                        buft.at[0], sem_c)

                o = pl.multiple_of(OSC + c * CWA, 128)

                def rows_body(r, _4):
                    count_segment(bufs[k], r, g * 8 + r, o, CVA)
                    return 0
                lax.fori_loop(0, 8, rows_body, 0)
            return 0
        lax.fori_loop(0, NA // 2, pair_body, 0)

        wait_dma(bufr, CRG)
        pltpu.make_async_copy(mat_hbm.at[pl.ds(0, 8), pl.ds(OTL, CTL)],
                              buft.at[0], sem_c).wait()
        rs_n = pl.multiple_of(jnp.minimum(base + (g + 1) * 8, N - 8), 8)
        pltpu.async_copy(mat_hbm.at[pl.ds(rs_n, 8), pl.ds(OSC, CWA)],
                         bufa0, sem_c)

        def rows_bodyr(r, _6):
            count_segment(bufr, r, g * 8 + r, ORG, CVR)
            count_segment(buft.at[0], r, g * 8 + r, OTL, CVT)
            return 0
        lax.fori_loop(0, 8, rows_bodyr, 0)

        def fin_body(r, _7):
            j = g * 8 + r
            rank = (jnp.sum(accb[pl.ds(j * L, L)])
                    + jnp.max(pcntb[pl.ds(j * L, L)]))
            rblock[j, pl.ds(0, L)] = jnp.full((L,), rank, jnp.int32)
            return 0
        lax.fori_loop(0, 8, fin_body, 0)
        return 0

    lax.fori_loop(0, G, group_body, 0)
    wait_dma(bufa0, CWA)
    pltpu.sync_copy(rblock, out_hbm.at[pl.ds(base, 8 * G), :])


def _tc_body(x_ref, tv_ref, tg_ref, o_ref):
    x = x_ref[...]
    tv = tv_ref[...]
    tg = tg_ref[...]
    ci = lax.broadcasted_iota(jnp.int32, x.shape, 1)
    m = (x > tv) | ((x == tv) & (ci < tg))
    o_ref[...] = jnp.sum(m.astype(jnp.int32), axis=1, keepdims=True)


@jax.jit
def kernel(logits, targets):
    mat = logits.reshape(N, V)
    tflat = targets.reshape(-1).astype(jnp.int32)
    mesh = plsc.VectorSubcoreMesh(core_axis_name="c", subcore_axis_name="s")
    scp = pltpu.CompilerParams(needs_layout_passes=False)

    tvals = pl.kernel(
        _gather_body,
        out_type=jax.ShapeDtypeStruct((N * L,), jnp.float32),
        mesh=mesh,
        scratch_types=[
            pltpu.VMEM((8 * G,), jnp.int32),
            pltpu.VMEM((8 * G, 8, 128), jnp.float32),
            pltpu.VMEM((8 * G * L,), jnp.float32),
            pltpu.SemaphoreType.DMA,
        ],
        compiler_params=scp,
    )(mat, tflat)

    rank_sc = pl.kernel(
        _count_body,
        out_type=jax.ShapeDtypeStruct((N, L), jnp.int32),
        mesh=mesh,
        scratch_types=[
            pltpu.VMEM((8 * G,), jnp.int32),
            pltpu.VMEM((8 * G * L,), jnp.float32),
            pltpu.VMEM((8 * G * L,), jnp.int32),
            pltpu.VMEM((8 * G * L,), jnp.int32),
            pltpu.VMEM((8, CWA), jnp.float32),
            pltpu.VMEM((8, CWA), jnp.float32),
            pltpu.VMEM((8, CRG), jnp.float32),
            pltpu.VMEM((1, 8, CTL), jnp.float32),
            pltpu.VMEM((8 * G, L), jnp.int32),
            pltpu.SemaphoreType.DMA,
        ],
        compiler_params=scp,
    )(mat, tflat, tvals)

    tv2 = tvals.reshape(N, L)[:, :1]
    tg2 = tflat[:, None]
    rank_tc = pl.pallas_call(
        _tc_body,
        grid=(N // 32,),
        in_specs=[
            pl.BlockSpec((32, OSC), lambda i: (i, 0)),
            pl.BlockSpec((32, 1), lambda i: (i, 0)),
            pl.BlockSpec((32, 1), lambda i: (i, 0)),
        ],
        out_specs=pl.BlockSpec((32, 1), lambda i: (i, 0)),
        out_shape=jax.ShapeDtypeStruct((N, 1), jnp.int32),
    )(mat, tv2, tg2)

    rank = rank_sc[:, 0] + rank_tc[:, 0]
    valid = tflat != IGN
    hit = (rank < TOPK) & valid
    correct = hit.sum().astype(jnp.float32)
    vcnt = valid.sum().astype(jnp.float32)
    acc = correct / jnp.maximum(vcnt, 1.0)
    return jnp.where(vcnt == 0, jnp.float32(0.0), acc).astype(jnp.float32)


# speedup vs baseline: 1.0033x; 1.0033x over previous
import jax
import jax.numpy as jnp
from jax import lax
from jax.experimental import pallas as pl
from jax.experimental.pallas import tpu as pltpu
from jax.experimental.pallas import tpu_sc as plsc

TOPK = 5
IGN = -100
V = 100000
N = 1024
L = 16
NW = 32
G = 4
OSC = 49280
CWA = 6272
NA = 8
ORG = OSC + CWA * NA
CRG = 512
OTL = ORG + CRG
CTL = V - OTL
CVA = CWA // L
CVR = CRG // L
CVT = CTL // L
UNROLL = 8


def _gather_body(mat_hbm, targ_hbm, tv_hbm, targv, wbuf, stage, sem_w):
    wid = lax.axis_index("s") * 2 + lax.axis_index("c")
    base = wid * (8 * G)
    pltpu.sync_copy(targ_hbm.at[pl.ds(base, 8 * G)], targv)

    for j in range(8 * G):
        v16 = targv[pl.ds((j // L) * L, L)]
        tj = jnp.clip(v16[j % L], 0, V - 1)
        a128 = pl.multiple_of((tj // 128) * 128, 128)
        rs = pl.multiple_of(base + (j // 8) * 8, 8)
        pltpu.async_copy(mat_hbm.at[pl.ds(rs, 8), pl.ds(a128, 128)],
                         wbuf.at[j], sem_w)
    for j in range(8 * G):
        pltpu.make_async_copy(mat_hbm.at[pl.ds(0, 8), pl.ds(0, 128)],
                              wbuf.at[j], sem_w).wait()
    for j in range(8 * G):
        tidx16 = plsc.load_gather(targv, [jnp.full((L,), j, jnp.int32)])
        tcv = jnp.clip(tidx16, 0, V - 1)
        tval16 = plsc.load_gather(
            wbuf, [jnp.full((L,), j, jnp.int32),
                   jnp.full((L,), j % 8, jnp.int32), tcv % 128])
        stage[pl.ds(j * L, L)] = tval16
    pltpu.sync_copy(stage, tv_hbm.at[pl.ds(base * L, 8 * G * L)])


def _count_body(mat_hbm, targ_hbm, tv_hbm, out_hbm, targv, tvv, accb, pcntb,
                bufa0, bufa1, bufr, buft, rblock, sem_c):
    wid = lax.axis_index("s") * 2 + lax.axis_index("c")
    base = wid * (8 * G)

    pltpu.sync_copy(targ_hbm.at[pl.ds(base, 8 * G)], targv)
    pltpu.sync_copy(tv_hbm.at[pl.ds(base * L, 8 * G * L)], tvv)

    lane = lax.iota(jnp.int32, L)
    zero16 = jnp.zeros((L,), jnp.int32)
    bufs = (bufa0, bufa1)
    for j in range(8 * G):
        accb[pl.ds(j * L, L)] = zero16
        pcntb[pl.ds(j * L, L)] = zero16

    pltpu.async_copy(mat_hbm.at[pl.ds(base, 8), pl.ds(OSC, CWA)], bufa0, sem_c)

    def wait_dma(buf, w):
        pltpu.make_async_copy(mat_hbm.at[pl.ds(0, 8), pl.ds(0, w)],
                              buf, sem_c).wait()

    def count_segment(buf, r, j, o, cv):
        tval16 = tvv[pl.ds(j * L, L)]
        tidx16 = plsc.load_gather(targv, [jnp.full((L,), j, jnp.int32)])
        tc = jnp.max(jnp.clip(tidx16, 0, V - 1))
        s = jnp.clip(tc - o, 0, cv * L)
        fs = s // L
        pcnt = pcntb[pl.ds(j * L, L)]
        acc = accb[pl.ds(j * L, L)]

        @plsc.parallel_loop(0, fs, unroll=UNROLL, carry=pcnt)
        def ge_loop(i, a):
            x = buf[r, pl.ds(i * L, L)]
            return a + plsc.all_reduce_population_count(x >= tval16)
        pcnt = ge_loop

        fm = jnp.minimum(fs, cv - 1)
        x = buf[r, pl.ds(fm * L, L)]
        posv = jnp.full((L,), o + fm * L, jnp.int32) + lane
        m = (x > tval16) | ((x == tval16) & (posv < tidx16))
        m = m & (jnp.full((L,), fs, jnp.int32) < cv)
        acc = acc + jnp.where(m, 1, 0).astype(jnp.int32)

        @plsc.parallel_loop(fs + 1, cv, unroll=UNROLL, carry=pcnt)
        def gt_loop(i, a):
            x = buf[r, pl.ds(i * L, L)]
            return a + plsc.all_reduce_population_count(x > tval16)
        pcnt = gt_loop

        pcntb[pl.ds(j * L, L)] = pcnt
        accb[pl.ds(j * L, L)] = acc

    def group_body(g, _):
        rs_g = pl.multiple_of(base + g * 8, 8)

        def pair_body(p, _2):
            for k in range(2):
                c = 2 * p + k
                wait_dma(bufs[k], CWA)
                o_next = pl.multiple_of(OSC + (c + 1) * CWA, 128)

                @pl.when(c < NA - 1)
                def _3():
                    pltpu.async_copy(
                        mat_hbm.at[pl.ds(rs_g, 8), pl.ds(o_next, CWA)],
                        bufs[1 - k], sem_c)

                @pl.when(c == NA - 1)
                def _3b():
                    pltpu.async_copy(
                        mat_hbm.at[pl.ds(rs_g, 8), pl.ds(ORG, CRG)],
                        bufr, sem_c)
                    pltpu.async_copy(
                        mat_hbm.at[pl.ds(rs_g, 8), pl.ds(OTL, CTL)],
                        buft.at[0], sem_c)

                o = pl.multiple_of(OSC + c * CWA, 128)

                def rows_body(r, _4):
                    count_segment(bufs[k], r, g * 8 + r, o, CVA)
                    return 0
                lax.fori_loop(0, 8, rows_body, 0)
            return 0
        lax.fori_loop(0, NA // 2, pair_body, 0)

        wait_dma(bufr, CRG)
        pltpu.make_async_copy(mat_hbm.at[pl.ds(0, 8), pl.ds(OTL, CTL)],
                              buft.at[0], sem_c).wait()
        rs_n = pl.multiple_of(jnp.minimum(base + (g + 1) * 8, N - 8), 8)
        pltpu.async_copy(mat_hbm.at[pl.ds(rs_n, 8), pl.ds(OSC, CWA)],
                         bufa0, sem_c)

        def rows_bodyr(r, _6):
            count_segment(bufr, r, g * 8 + r, ORG, CVR)
            count_segment(buft.at[0], r, g * 8 + r, OTL, CVT)
            return 0
        lax.fori_loop(0, 8, rows_bodyr, 0)

        def fin_body(r, _7):
            j = g * 8 + r
            rank = (jnp.sum(accb[pl.ds(j * L, L)])
                    + jnp.max(pcntb[pl.ds(j * L, L)]))
            rblock[j, pl.ds(0, L)] = jnp.full((L,), rank, jnp.int32)
            return 0
        lax.fori_loop(0, 8, fin_body, 0)
        return 0

    lax.fori_loop(0, G, group_body, 0)
    wait_dma(bufa0, CWA)
    pltpu.sync_copy(rblock, out_hbm.at[pl.ds(base, 8 * G), :])


def _tc_body(x_ref, tv_ref, tg_ref, o_ref):
    x = x_ref[...]
    tv = tv_ref[...]
    tg = tg_ref[...]
    ci = lax.broadcasted_iota(jnp.int32, x.shape, 1)
    m = (x > tv) | ((x == tv) & (ci < tg))
    o_ref[...] = jnp.sum(m.astype(jnp.int32), axis=1, keepdims=True)


@jax.jit
def kernel(logits, targets):
    mat = logits.reshape(N, V)
    tflat = targets.reshape(-1).astype(jnp.int32)
    mesh = plsc.VectorSubcoreMesh(core_axis_name="c", subcore_axis_name="s")
    scp = pltpu.CompilerParams(needs_layout_passes=False)

    tvals = pl.kernel(
        _gather_body,
        out_type=jax.ShapeDtypeStruct((N * L,), jnp.float32),
        mesh=mesh,
        scratch_types=[
            pltpu.VMEM((8 * G,), jnp.int32),
            pltpu.VMEM((8 * G, 8, 128), jnp.float32),
            pltpu.VMEM((8 * G * L,), jnp.float32),
            pltpu.SemaphoreType.DMA,
        ],
        compiler_params=scp,
    )(mat, tflat)

    rank_sc = pl.kernel(
        _count_body,
        out_type=jax.ShapeDtypeStruct((N, L), jnp.int32),
        mesh=mesh,
        scratch_types=[
            pltpu.VMEM((8 * G,), jnp.int32),
            pltpu.VMEM((8 * G * L,), jnp.float32),
            pltpu.VMEM((8 * G * L,), jnp.int32),
            pltpu.VMEM((8 * G * L,), jnp.int32),
            pltpu.VMEM((8, CWA), jnp.float32),
            pltpu.VMEM((8, CWA), jnp.float32),
            pltpu.VMEM((8, CRG), jnp.float32),
            pltpu.VMEM((1, 8, CTL), jnp.float32),
            pltpu.VMEM((8 * G, L), jnp.int32),
            pltpu.SemaphoreType.DMA,
        ],
        compiler_params=scp,
    )(mat, tflat, tvals)

    tv2 = tvals.reshape(N, L)[:, :1]
    tg2 = tflat[:, None]
    rank_tc = pl.pallas_call(
        _tc_body,
        grid=(N // 64,),
        in_specs=[
            pl.BlockSpec((64, OSC), lambda i: (i, 0)),
            pl.BlockSpec((64, 1), lambda i: (i, 0)),
            pl.BlockSpec((64, 1), lambda i: (i, 0)),
        ],
        out_specs=pl.BlockSpec((64, 1), lambda i: (i, 0)),
        out_shape=jax.ShapeDtypeStruct((N, 1), jnp.int32),
    )(mat, tv2, tg2)

    rank = rank_sc[:, 0] + rank_tc[:, 0]
    valid = tflat != IGN
    hit = (rank < TOPK) & valid
    correct = hit.sum().astype(jnp.float32)
    vcnt = valid.sum().astype(jnp.float32)
    acc = correct / jnp.maximum(vcnt, 1.0)
    return jnp.where(vcnt == 0, jnp.float32(0.0), acc).astype(jnp.float32)


# speedup vs baseline: 1.0240x; 1.0207x over previous
import jax
import jax.numpy as jnp
from jax import lax
from jax.experimental import pallas as pl
from jax.experimental.pallas import tpu as pltpu
from jax.experimental.pallas import tpu_sc as plsc

TOPK = 5
IGN = -100
V = 100000
N = 1024
L = 16
NW = 32
G = 4
OSC = 52480
CWA = 5888
NA = 8
ORG = OSC + CWA * NA
CRG = 384
OTL = ORG + CRG
CTL = V - OTL
CVA = CWA // L
CVR = CRG // L
CVT = CTL // L
UNROLL = 8


def _gather_body(mat_hbm, targ_hbm, tv_hbm, targv, wbuf, stage, sem_w):
    wid = lax.axis_index("s") * 2 + lax.axis_index("c")
    base = wid * (8 * G)
    pltpu.sync_copy(targ_hbm.at[pl.ds(base, 8 * G)], targv)

    for j in range(8 * G):
        v16 = targv[pl.ds((j // L) * L, L)]
        tj = jnp.clip(v16[j % L], 0, V - 1)
        a128 = pl.multiple_of((tj // 128) * 128, 128)
        rs = pl.multiple_of(base + (j // 8) * 8, 8)
        pltpu.async_copy(mat_hbm.at[pl.ds(rs, 8), pl.ds(a128, 128)],
                         wbuf.at[j], sem_w)
    for j in range(8 * G):
        pltpu.make_async_copy(mat_hbm.at[pl.ds(0, 8), pl.ds(0, 128)],
                              wbuf.at[j], sem_w).wait()
    for j in range(8 * G):
        tidx16 = plsc.load_gather(targv, [jnp.full((L,), j, jnp.int32)])
        tcv = jnp.clip(tidx16, 0, V - 1)
        tval16 = plsc.load_gather(
            wbuf, [jnp.full((L,), j, jnp.int32),
                   jnp.full((L,), j % 8, jnp.int32), tcv % 128])
        stage[pl.ds(j * L, L)] = tval16
    pltpu.sync_copy(stage, tv_hbm.at[pl.ds(base * L, 8 * G * L)])


def _count_body(mat_hbm, targ_hbm, tv_hbm, out_hbm, targv, tvv, accb, pcntb,
                bufa0, bufa1, bufr, buft, rblock, sem_c):
    wid = lax.axis_index("s") * 2 + lax.axis_index("c")
    base = wid * (8 * G)

    pltpu.sync_copy(targ_hbm.at[pl.ds(base, 8 * G)], targv)
    pltpu.sync_copy(tv_hbm.at[pl.ds(base * L, 8 * G * L)], tvv)

    lane = lax.iota(jnp.int32, L)
    zero16 = jnp.zeros((L,), jnp.int32)
    bufs = (bufa0, bufa1)
    for j in range(8 * G):
        accb[pl.ds(j * L, L)] = zero16
        pcntb[pl.ds(j * L, L)] = zero16

    pltpu.async_copy(mat_hbm.at[pl.ds(base, 8), pl.ds(OSC, CWA)], bufa0, sem_c)

    def wait_dma(buf, w):
        pltpu.make_async_copy(mat_hbm.at[pl.ds(0, 8), pl.ds(0, w)],
                              buf, sem_c).wait()

    def count_segment(buf, r, j, o, cv):
        tval16 = tvv[pl.ds(j * L, L)]
        tidx16 = plsc.load_gather(targv, [jnp.full((L,), j, jnp.int32)])
        tc = jnp.max(jnp.clip(tidx16, 0, V - 1))
        s = jnp.clip(tc - o, 0, cv * L)
        fs = s // L
        pcnt = pcntb[pl.ds(j * L, L)]
        acc = accb[pl.ds(j * L, L)]

        @plsc.parallel_loop(0, fs, unroll=UNROLL, carry=pcnt)
        def ge_loop(i, a):
            x = buf[r, pl.ds(i * L, L)]
            return a + plsc.all_reduce_population_count(x >= tval16)
        pcnt = ge_loop

        fm = jnp.minimum(fs, cv - 1)
        x = buf[r, pl.ds(fm * L, L)]
        posv = jnp.full((L,), o + fm * L, jnp.int32) + lane
        m = (x > tval16) | ((x == tval16) & (posv < tidx16))
        m = m & (jnp.full((L,), fs, jnp.int32) < cv)
        acc = acc + jnp.where(m, 1, 0).astype(jnp.int32)

        @plsc.parallel_loop(fs + 1, cv, unroll=UNROLL, carry=pcnt)
        def gt_loop(i, a):
            x = buf[r, pl.ds(i * L, L)]
            return a + plsc.all_reduce_population_count(x > tval16)
        pcnt = gt_loop

        pcntb[pl.ds(j * L, L)] = pcnt
        accb[pl.ds(j * L, L)] = acc

    def group_body(g, _):
        rs_g = pl.multiple_of(base + g * 8, 8)

        def pair_body(p, _2):
            for k in range(2):
                c = 2 * p + k
                wait_dma(bufs[k], CWA)
                o_next = pl.multiple_of(OSC + (c + 1) * CWA, 128)

                @pl.when(c < NA - 1)
                def _3():
                    pltpu.async_copy(
                        mat_hbm.at[pl.ds(rs_g, 8), pl.ds(o_next, CWA)],
                        bufs[1 - k], sem_c)

                @pl.when(c == NA - 1)
                def _3b():
                    pltpu.async_copy(
                        mat_hbm.at[pl.ds(rs_g, 8), pl.ds(ORG, CRG)],
                        bufr, sem_c)
                    pltpu.async_copy(
                        mat_hbm.at[pl.ds(rs_g, 8), pl.ds(OTL, CTL)],
                        buft.at[0], sem_c)

                o = pl.multiple_of(OSC + c * CWA, 128)

                def rows_body(r, _4):
                    count_segment(bufs[k], r, g * 8 + r, o, CVA)
                    return 0
                lax.fori_loop(0, 8, rows_body, 0)
            return 0
        lax.fori_loop(0, NA // 2, pair_body, 0)

        wait_dma(bufr, CRG)
        pltpu.make_async_copy(mat_hbm.at[pl.ds(0, 8), pl.ds(OTL, CTL)],
                              buft.at[0], sem_c).wait()
        rs_n = pl.multiple_of(jnp.minimum(base + (g + 1) * 8, N - 8), 8)
        pltpu.async_copy(mat_hbm.at[pl.ds(rs_n, 8), pl.ds(OSC, CWA)],
                         bufa0, sem_c)

        def rows_bodyr(r, _6):
            count_segment(bufr, r, g * 8 + r, ORG, CVR)
            count_segment(buft.at[0], r, g * 8 + r, OTL, CVT)
            return 0
        lax.fori_loop(0, 8, rows_bodyr, 0)

        def fin_body(r, _7):
            j = g * 8 + r
            rank = (jnp.sum(accb[pl.ds(j * L, L)])
                    + jnp.max(pcntb[pl.ds(j * L, L)]))
            rblock[j, pl.ds(0, L)] = jnp.full((L,), rank, jnp.int32)
            return 0
        lax.fori_loop(0, 8, fin_body, 0)
        return 0

    lax.fori_loop(0, G, group_body, 0)
    wait_dma(bufa0, CWA)
    pltpu.sync_copy(rblock, out_hbm.at[pl.ds(base, 8 * G), :])


def _tc_body(x_ref, tv_ref, tg_ref, o_ref):
    x = x_ref[...]
    tv = tv_ref[...]
    tg = tg_ref[...]
    ci = lax.broadcasted_iota(jnp.int32, x.shape, 1)
    m = (x > tv) | ((x == tv) & (ci < tg))
    o_ref[...] = jnp.sum(m.astype(jnp.int32), axis=1, keepdims=True)


@jax.jit
def kernel(logits, targets):
    mat = logits.reshape(N, V)
    tflat = targets.reshape(-1).astype(jnp.int32)
    mesh = plsc.VectorSubcoreMesh(core_axis_name="c", subcore_axis_name="s")
    scp = pltpu.CompilerParams(needs_layout_passes=False)

    tvals = pl.kernel(
        _gather_body,
        out_type=jax.ShapeDtypeStruct((N * L,), jnp.float32),
        mesh=mesh,
        scratch_types=[
            pltpu.VMEM((8 * G,), jnp.int32),
            pltpu.VMEM((8 * G, 8, 128), jnp.float32),
            pltpu.VMEM((8 * G * L,), jnp.float32),
            pltpu.SemaphoreType.DMA,
        ],
        compiler_params=scp,
    )(mat, tflat)

    rank_sc = pl.kernel(
        _count_body,
        out_type=jax.ShapeDtypeStruct((N, L), jnp.int32),
        mesh=mesh,
        scratch_types=[
            pltpu.VMEM((8 * G,), jnp.int32),
            pltpu.VMEM((8 * G * L,), jnp.float32),
            pltpu.VMEM((8 * G * L,), jnp.int32),
            pltpu.VMEM((8 * G * L,), jnp.int32),
            pltpu.VMEM((8, CWA), jnp.float32),
            pltpu.VMEM((8, CWA), jnp.float32),
            pltpu.VMEM((8, CRG), jnp.float32),
            pltpu.VMEM((1, 8, CTL), jnp.float32),
            pltpu.VMEM((8 * G, L), jnp.int32),
            pltpu.SemaphoreType.DMA,
        ],
        compiler_params=scp,
    )(mat, tflat, tvals)

    tv2 = tvals.reshape(N, L)[:, :1]
    tg2 = tflat[:, None]
    rank_tc = pl.pallas_call(
        _tc_body,
        grid=(N // 64,),
        in_specs=[
            pl.BlockSpec((64, OSC), lambda i: (i, 0)),
            pl.BlockSpec((64, 1), lambda i: (i, 0)),
            pl.BlockSpec((64, 1), lambda i: (i, 0)),
        ],
        out_specs=pl.BlockSpec((64, 1), lambda i: (i, 0)),
        out_shape=jax.ShapeDtypeStruct((N, 1), jnp.int32),
    )(mat, tv2, tg2)

    rank = rank_sc[:, 0] + rank_tc[:, 0]
    valid = tflat != IGN
    hit = (rank < TOPK) & valid
    correct = hit.sum().astype(jnp.float32)
    vcnt = valid.sum().astype(jnp.float32)
    acc = correct / jnp.maximum(vcnt, 1.0)
    return jnp.where(vcnt == 0, jnp.float32(0.0), acc).astype(jnp.float32)
